# baseline (speedup 1.0000x reference)
import functools

import jax
import jax.numpy as jnp
from jax import lax
from jax.experimental import pallas as pl
from jax.experimental.pallas import tpu as pltpu
from jax.experimental.pallas import tpu_sc as plsc

N = 524288
M = 32
L = 64
NC = 2
NS = 16
NW = NC * NS
TPW = N // NW
C = 512
NCHUNK = TPW // C
RPT = C // L
SPC = C // 128
TPC = SPC * 1024


def _sc_body(tok_hbm, emb_hbm, b2_hbm, ap_hbm, out_hbm,
             idx_v, rows_v, outf_v, b2_v, ap_v, gsem, osem):
    wid = lax.axis_index("s") * NC + lax.axis_index("c")
    pltpu.sync_copy(b2_hbm, b2_v)
    pltpu.sync_copy(ap_hbm, ap_v)
    pltpu.sync_copy(
        tok_hbm.at[pl.ds(pl.multiple_of(wid * (TPW // 128), 8), TPW // 128)],
        idx_v)

    def gathers(c, b):
        for s in range(SPC):
            pltpu.async_copy(emb_hbm.at[idx_v.at[c * SPC + s]],
                             rows_v.at[b, pl.ds(s * 128, 128)], gsem.at[b])

    def wait_gathers(c, b):
        for s in range(SPC):
            pltpu.make_async_copy(emb_hbm.at[idx_v.at[c * SPC + s]],
                                  rows_v.at[b, pl.ds(s * 128, 128)],
                                  gsem.at[b]).wait()

    def out_copy(c, b):
        b0 = wid * (TPW // 128) + c * SPC
        copies = []
        for a in range(4):
            dst = pl.multiple_of((a * (N // 128) + b0) * 1024, 8)
            copies.append(pltpu.make_async_copy(
                outf_v.at[b, pl.ds(a * TPC, TPC)],
                out_hbm.at[pl.ds(dst, TPC)],
                osem.at[b]))
        return copies

    lane_c = lax.iota(jnp.int32, 16)
    rots = [(lane_c + (1 << k)) % 16 for k in range(4)]

    def _lanesum(v):
        for rk in rots:
            v = v + jnp.take(v, rk)
        return v

    def compute(b):
        def gbody(g, carry2):
            t0 = g * 16
            jb = (g % 4) * 16
            chains = [jnp.zeros((16,), jnp.float32) for _ in range(4)]
            for r in range(16):
                t = t0 + r
                j = jb + r
                xlo = rows_v[b, t, 0:16]
                xhi = rows_v[b, t, 16:32]
                s = _lanesum(b2_v[j, 0:16] * xlo + b2_v[j, 16:32] * xhi)
                q = r // 4
                chains[q] = jnp.where(lane_c == r, s, chains[q])
            s01 = jnp.where(lane_c < 4, chains[0], chains[1])
            s23 = jnp.where(lane_c < 12, chains[2], chains[3])
            s_vec = jnp.where(lane_c < 8, s01, s23)
            p = g % 4
            soff = (t0 // 128) * 1024 + (t0 % 128)
            for m in range(M):
                off = soff + (m // 8) * TPC + (m % 8) * 128
                outf_v[b, pl.ds(pl.multiple_of(off, 8), 16)] = \
                    ap_v[p, m, :] * s_vec
            return carry2

        lax.fori_loop(0, C // 16, gbody, 0)

    def step(c, b, pre_c, pre_cond, wait_cond):
        if pre_cond is None:
            gathers(pre_c, 1 - b)
        else:
            @pl.when(pre_cond)
            def _():
                gathers(pre_c, 1 - b)
        wait_gathers(c, b)

        @pl.when(wait_cond)
        def _():
            for cp in out_copy(c - 2, b):
                cp.wait()

        compute(b)
        for cp in out_copy(c, b):
            cp.start()

    gathers(0, 0)

    def pair(co, carry):
        c = co * 2
        step(c, 0, pre_c=c + 1, pre_cond=None, wait_cond=co >= 1)
        step(c + 1, 1, pre_c=c + 2, pre_cond=co < NCHUNK // 2 - 1,
             wait_cond=co >= 1)
        return carry

    lax.fori_loop(0, NCHUNK // 2, pair, 0)

    for cp in out_copy(NCHUNK - 2, 0):
        cp.wait()
    for cp in out_copy(NCHUNK - 1, 1):
        cp.wait()


@functools.partial(jax.jit, static_argnames=())
def kernel(token_indices, embedding, Acoeff, Bbasis):
    tok = token_indices.astype(jnp.int32).reshape(N // 128, 128)
    ap = Acoeff.reshape(M, 4, 16).transpose(1, 0, 2)
    mesh = plsc.VectorSubcoreMesh(core_axis_name="c", subcore_axis_name="s",
                                  num_cores=NC, num_subcores=NS)
    f = pl.kernel(
        _sc_body,
        out_type=jax.ShapeDtypeStruct((N * M,), jnp.float32),
        mesh=mesh,
        compiler_params=pltpu.CompilerParams(needs_layout_passes=False,
                                             use_tc_tiling_on_sc=False),
        scratch_types=[
            pltpu.VMEM((TPW // 128, 128), jnp.int32),
            pltpu.VMEM((2, C, M), jnp.float32),
            pltpu.VMEM((2, C * M), jnp.float32),
            pltpu.VMEM((L, M), jnp.float32),
            pltpu.VMEM((4, M, 16), jnp.float32),
            pltpu.SemaphoreType.DMA((2,)),
            pltpu.SemaphoreType.DMA((2,)),
        ],
    )
    flat = f(tok, embedding, Bbasis, ap)
    return flat.reshape(4, N // 128, 8, 128).transpose(1, 3, 0, 2).reshape(N, M)


# speedup vs baseline: 1.0765x; 1.0765x over previous
import functools

import jax
import jax.numpy as jnp
from jax import lax
from jax.experimental import pallas as pl
from jax.experimental.pallas import tpu as pltpu
from jax.experimental.pallas import tpu_sc as plsc

N = 524288
M = 32
L = 64
NC = 2
NS = 16
NW = NC * NS
TPW = N // NW
C = 512
NCHUNK = TPW // C
RPT = C // L
SPC = C // 128
TPC = SPC * 1024


def _sc_body(tok_hbm, emb_hbm, b2_hbm, ap_hbm, out_hbm,
             idx_v, rows_v, outf_v, b2_v, ap_v, gsem, osem):
    wid = lax.axis_index("s") * NC + lax.axis_index("c")
    pltpu.sync_copy(b2_hbm, b2_v)
    pltpu.sync_copy(ap_hbm, ap_v)
    pltpu.sync_copy(
        tok_hbm.at[pl.ds(pl.multiple_of(wid * (TPW // 128), 8), TPW // 128)],
        idx_v)

    def gathers(c, b):
        for s in range(SPC):
            pltpu.async_copy(emb_hbm.at[idx_v.at[c * SPC + s]],
                             rows_v.at[b, pl.ds(s * 128, 128)], gsem.at[b])

    def wait_gathers(c, b):
        for s in range(SPC):
            pltpu.make_async_copy(emb_hbm.at[idx_v.at[c * SPC + s]],
                                  rows_v.at[b, pl.ds(s * 128, 128)],
                                  gsem.at[b]).wait()

    def out_copy(c, b):
        b0 = wid * (TPW // 128) + c * SPC
        copies = []
        for a in range(4):
            dst = pl.multiple_of((a * (N // 128) + b0) * 1024, 8)
            copies.append(pltpu.make_async_copy(
                outf_v.at[b, pl.ds(a * TPC, TPC)],
                out_hbm.at[pl.ds(dst, TPC)],
                osem.at[b]))
        return copies

    lane_c = lax.iota(jnp.int32, 16)

    def compute(b):
        def gbody(g, carry2):
            t0 = g * 16
            jb = (g % 4) * 16
            s_vec = jnp.zeros((16,), jnp.float32)
            for r in range(16):
                t = t0 + r
                j = jb + r
                xlo = rows_v[b, t, 0:16]
                xhi = rows_v[b, t, 16:32]
                s = jnp.sum(b2_v[j, 0:16] * xlo + b2_v[j, 16:32] * xhi)
                s_vec = jnp.where(lane_c == r, s, s_vec)
            p = g % 4
            soff = (t0 // 128) * 1024 + (t0 % 128)
            for m in range(M):
                off = soff + (m // 8) * TPC + (m % 8) * 128
                outf_v[b, pl.ds(pl.multiple_of(off, 8), 16)] = \
                    ap_v[p, m, :] * s_vec
            return carry2

        lax.fori_loop(0, C // 16, gbody, 0)

    def step(c, b, pre_c, pre_cond, wait_cond):
        if pre_cond is None:
            gathers(pre_c, 1 - b)
        else:
            @pl.when(pre_cond)
            def _():
                gathers(pre_c, 1 - b)
        wait_gathers(c, b)

        @pl.when(wait_cond)
        def _():
            for cp in out_copy(c - 2, b):
                cp.wait()

        compute(b)
        for cp in out_copy(c, b):
            cp.start()

    gathers(0, 0)

    def pair(co, carry):
        c = co * 2
        step(c, 0, pre_c=c + 1, pre_cond=None, wait_cond=co >= 1)
        step(c + 1, 1, pre_c=c + 2, pre_cond=co < NCHUNK // 2 - 1,
             wait_cond=co >= 1)
        return carry

    lax.fori_loop(0, NCHUNK // 2, pair, 0)

    for cp in out_copy(NCHUNK - 2, 0):
        cp.wait()
    for cp in out_copy(NCHUNK - 1, 1):
        cp.wait()


@functools.partial(jax.jit, static_argnames=())
def kernel(token_indices, embedding, Acoeff, Bbasis):
    tok = token_indices.astype(jnp.int32).reshape(N // 128, 128)
    ap = Acoeff.reshape(M, 4, 16).transpose(1, 0, 2)
    mesh = plsc.VectorSubcoreMesh(core_axis_name="c", subcore_axis_name="s",
                                  num_cores=NC, num_subcores=NS)
    f = pl.kernel(
        _sc_body,
        out_type=jax.ShapeDtypeStruct((N * M,), jnp.float32),
        mesh=mesh,
        compiler_params=pltpu.CompilerParams(needs_layout_passes=False,
                                             use_tc_tiling_on_sc=False),
        scratch_types=[
            pltpu.VMEM((TPW // 128, 128), jnp.int32),
            pltpu.VMEM((2, C, M), jnp.float32),
            pltpu.VMEM((2, C * M), jnp.float32),
            pltpu.VMEM((L, M), jnp.float32),
            pltpu.VMEM((4, M, 16), jnp.float32),
            pltpu.SemaphoreType.DMA((2,)),
            pltpu.SemaphoreType.DMA((2,)),
        ],
    )
    flat = f(tok, embedding, Bbasis, ap)
    return flat.reshape(4, N // 128, 8, 128).transpose(1, 3, 0, 2).reshape(N, M)
